# baseline (speedup 1.0000x reference)
import functools

import jax
import jax.numpy as jnp
from jax import lax
from jax.experimental import pallas as pl
from jax.experimental.pallas import tpu as pltpu
from jax.experimental.pallas import tpu_sc as plsc

NTILES = 16
NCORES = 2
GROUP = 96
NBUF = 3
NIDX = 4


def _sc_aggregate(x, sd, z2, z1, ones, n_rows, half):
    g_steps = sd.shape[1]
    rows_pt = n_rows // NTILES
    mesh = plsc.VectorSubcoreMesh(core_axis_name="c", subcore_axis_name="s")

    @functools.partial(
        pl.kernel,
        mesh=mesh,
        out_type=[
            jax.ShapeDtypeStruct((NCORES, n_rows, half), jnp.float32),
            jax.ShapeDtypeStruct((NCORES, n_rows), jnp.float32),
        ],
        scratch_types=[
            pltpu.VMEM((NIDX, 2, GROUP), jnp.int32),
            pltpu.VMEM((NBUF, GROUP, half), jnp.float32),
            pltpu.VMEM((GROUP,), jnp.float32),
            pltpu.VMEM((rows_pt,), jnp.float32),
            pltpu.VMEM_SHARED((n_rows, half), jnp.float32),
            pltpu.VMEM_SHARED((n_rows,), jnp.float32),
            pltpu.SemaphoreType.DMA((NIDX,)),
            pltpu.SemaphoreType.DMA((NBUF,)),
            pltpu.SemaphoreType.DMA((NBUF,)),
            pltpu.SemaphoreType.DMA((NBUF,)),
        ],
    )
    def sc_kernel(x_r, sd_r, z2_r, z1_r, ones_r,
                  out_sum, out_deg,
                  sd_v, rows, ones_v, deg_v, acc, dacc, isem, gsem, ssem, dsem):
        c = lax.axis_index("c")
        t = lax.axis_index("s")
        chunks = []
        off = 0
        while off < rows_pt:
            chunks.append((off, min(GROUP, rows_pt - off)))
            off += GROUP

        def run(x_half, out_half, out_deg_c, parity):
            base_r = t * rows_pt
            pltpu.sync_copy(z2_r, rows.at[0])
            for i, (o, sz) in enumerate(chunks):
                pltpu.async_copy(rows.at[0].at[pl.ds(0, sz)],
                                 acc.at[pl.ds(base_r + o, sz)],
                                 ssem.at[i % NBUF])
            for i, (o, sz) in enumerate(chunks):
                pltpu.make_async_copy(rows.at[0].at[pl.ds(0, sz)],
                                      acc.at[pl.ds(base_r + o, sz)],
                                      ssem.at[i % NBUF]).wait()
            pltpu.sync_copy(ones_r, ones_v)
            pltpu.sync_copy(z1_r, deg_v)
            pltpu.sync_copy(deg_v, dacc.at[pl.ds(base_r, rows_pt)])
            for j in range(NIDX):
                pltpu.async_copy(sd_r.at[t, j], sd_v.at[j], isem.at[j])
            for b in range(NBUF):
                pltpu.make_async_copy(sd_r.at[t, b], sd_v.at[b],
                                      isem.at[b]).wait()
                pltpu.async_copy(x_half.at[sd_v.at[b, 0]], rows.at[b],
                                 gsem.at[b])
            plsc.subcore_barrier()

            @pl.loop(0, g_steps, step=NBUF)
            def _(g0):
                for b in range(NBUF):
                    g = g0 + b
                    j = lax.rem(g, NIDX)
                    pltpu.make_async_copy(x_half.at[sd_v.at[j, 0]],
                                          rows.at[b], gsem.at[b]).wait()
                    pltpu.async_copy(rows.at[b], acc.at[sd_v.at[j, 1]],
                                     ssem.at[b], add=True)

                    @pl.when(lax.rem(g, 2) == parity)
                    def _():
                        pltpu.async_copy(ones_v, dacc.at[sd_v.at[j, 1]],
                                         dsem.at[b], add=True)
                for b in range(NBUF):
                    g = g0 + b
                    gn = g + NBUF
                    h = g + NIDX

                    @pl.when(gn < g_steps)
                    def _():
                        j = lax.rem(g, NIDX)
                        jn = lax.rem(gn, NIDX)
                        pltpu.make_async_copy(rows.at[b],
                                              acc.at[sd_v.at[j, 1]],
                                              ssem.at[b]).wait()

                        @pl.when(lax.rem(g, 2) == parity)
                        def _():
                            pltpu.make_async_copy(ones_v,
                                                  dacc.at[sd_v.at[j, 1]],
                                                  dsem.at[b]).wait()

                        @pl.when(h < g_steps)
                        def _():
                            pltpu.async_copy(sd_r.at[t, h], sd_v.at[j],
                                             isem.at[j])

                        pltpu.make_async_copy(sd_r.at[t, gn], sd_v.at[jn],
                                              isem.at[jn]).wait()
                        pltpu.async_copy(x_half.at[sd_v.at[jn, 0]],
                                         rows.at[b], gsem.at[b])

            for b in range(NBUF):
                pltpu.make_async_copy(rows.at[b], acc.at[sd_v.at[0, 1]],
                                      ssem.at[b]).wait()
                if (g_steps - NBUF + b) % 2 == parity:
                    pltpu.make_async_copy(ones_v, dacc.at[sd_v.at[0, 1]],
                                          dsem.at[b]).wait()
            plsc.subcore_barrier()
            for i, (o, sz) in enumerate(chunks):
                b = i % NBUF
                if i >= NBUF:
                    po, psz = chunks[i - NBUF]
                    pltpu.make_async_copy(
                        rows.at[b].at[pl.ds(0, psz)],
                        out_half.at[pl.ds(base_r + po, psz)],
                        gsem.at[b]).wait()
                pltpu.sync_copy(acc.at[pl.ds(base_r + o, sz)],
                                rows.at[b].at[pl.ds(0, sz)])
                pltpu.async_copy(rows.at[b].at[pl.ds(0, sz)],
                                 out_half.at[pl.ds(base_r + o, sz)],
                                 gsem.at[b])
            for i in range(max(0, len(chunks) - NBUF), len(chunks)):
                o, sz = chunks[i]
                pltpu.make_async_copy(rows.at[i % NBUF].at[pl.ds(0, sz)],
                                      out_half.at[pl.ds(base_r + o, sz)],
                                      gsem.at[i % NBUF]).wait()
            pltpu.sync_copy(dacc.at[pl.ds(base_r, rows_pt)], deg_v)
            pltpu.sync_copy(deg_v, out_deg_c.at[pl.ds(base_r, rows_pt)])

        @pl.when(c == 0)
        def _():
            run(x_r.at[:, pl.ds(0, half)], out_sum.at[0], out_deg.at[0], 0)

        @pl.when(c == 1)
        def _():
            run(x_r.at[:, pl.ds(half, half)], out_sum.at[1], out_deg.at[1], 1)

    return sc_kernel(x, sd, z2, z1, ones)


def _tc_combine(x, sum2, deg, wst, wnlo, wnhi, b2, blk):
    n, d_in = x.shape
    d_out = wst.shape[1]
    half = sum2.shape[2]

    def body(x_ref, lo_ref, hi_ref, d0_ref, d1_ref, wst_ref, wnlo_ref,
             wnhi_ref, b_ref, out_ref):
        r = 1.0 / jnp.maximum(d0_ref[0] + d1_ref[0], 1.0)
        acc = jnp.dot(x_ref[...], wst_ref[...],
                      preferred_element_type=jnp.float32)
        nb = jnp.dot(lo_ref[0], wnlo_ref[...],
                     preferred_element_type=jnp.float32)
        nb = nb + jnp.dot(hi_ref[0], wnhi_ref[...],
                          preferred_element_type=jnp.float32)
        out_ref[...] = acc + nb * r + b_ref[...]

    return pl.pallas_call(
        body,
        grid=(n // blk,),
        in_specs=[
            pl.BlockSpec((blk, d_in), lambda i: (i, 0)),
            pl.BlockSpec((1, blk, half), lambda i: (0, i, 0)),
            pl.BlockSpec((1, blk, half), lambda i: (1, i, 0)),
            pl.BlockSpec((1, blk, 1), lambda i: (0, i, 0)),
            pl.BlockSpec((1, blk, 1), lambda i: (1, i, 0)),
            pl.BlockSpec((d_in, d_out), lambda i: (0, 0)),
            pl.BlockSpec((half, d_out), lambda i: (0, 0)),
            pl.BlockSpec((half, d_out), lambda i: (0, 0)),
            pl.BlockSpec((1, d_out), lambda i: (0, 0)),
        ],
        out_specs=pl.BlockSpec((blk, d_out), lambda i: (i, 0)),
        out_shape=jax.ShapeDtypeStruct((n, d_out), jnp.float32),
    )(x, sum2, sum2, deg, deg, wst, wnlo, wnhi, b2)


def kernel(x, edge_index, W_self, W_neigh, b):
    n, d_in = x.shape
    e = edge_index.shape[1]
    half = d_in // 2

    ept = -(-e // (NTILES * GROUP * NBUF)) * (GROUP * NBUF)
    e_pad = NTILES * ept
    g_steps = ept // GROUP
    n_rows = (n // (NTILES * 16) + 1) * (NTILES * 16)
    n_dummy = n_rows - n

    src = edge_index[0].astype(jnp.int32)
    dst = edge_index[1].astype(jnp.int32)
    pad_ar = jnp.arange(e_pad - e, dtype=jnp.int32)
    srcr = jnp.concatenate([src, pad_ar % n]).reshape(NTILES, g_steps, GROUP)
    dstr = jnp.concatenate([dst, n + pad_ar % n_dummy]).reshape(
        NTILES, g_steps, GROUP)
    sd = jnp.stack([srcr, dstr], axis=2)

    z2 = jnp.zeros((GROUP, half), jnp.float32)
    z1 = jnp.zeros((n_rows // NTILES,), jnp.float32)
    ones = jnp.ones((GROUP,), jnp.float32)

    wst = W_self.T
    wnt = W_neigh.T
    wnlo = wnt[:half]
    wnhi = wnt[half:]
    b2 = b.reshape(1, -1)
    sum2, deg = _sc_aggregate(x, sd, z2, z1, ones, n_rows, half)
    deg = deg.reshape(NCORES, n_rows, 1)
    return _tc_combine(x, sum2, deg, wst, wnlo, wnhi, b2, blk=1000)


# speedup vs baseline: 1.0013x; 1.0013x over previous
import functools

import jax
import jax.numpy as jnp
from jax import lax
from jax.experimental import pallas as pl
from jax.experimental.pallas import tpu as pltpu
from jax.experimental.pallas import tpu_sc as plsc

NTILES = 16
NCORES = 2
GROUP = 96
NBUF = 3
NIDX = 4


def _sc_aggregate(x, sd, z2, z1, ones, n_rows, half):
    g_steps = sd.shape[1]
    rows_pt = n_rows // NTILES
    mesh = plsc.VectorSubcoreMesh(core_axis_name="c", subcore_axis_name="s")

    @functools.partial(
        pl.kernel,
        mesh=mesh,
        out_type=[
            jax.ShapeDtypeStruct((NCORES, n_rows, half), jnp.float32),
            jax.ShapeDtypeStruct((n_rows,), jnp.float32),
        ],
        scratch_types=[
            pltpu.VMEM((NIDX, 2, GROUP), jnp.int32),
            pltpu.VMEM((NBUF, GROUP, half), jnp.float32),
            pltpu.VMEM((GROUP,), jnp.float32),
            pltpu.VMEM((rows_pt,), jnp.float32),
            pltpu.VMEM_SHARED((n_rows, half), jnp.float32),
            pltpu.VMEM_SHARED((n_rows,), jnp.float32),
            pltpu.SemaphoreType.DMA((NIDX,)),
            pltpu.SemaphoreType.DMA((NBUF,)),
            pltpu.SemaphoreType.DMA((NBUF,)),
            pltpu.SemaphoreType.DMA((NBUF,)),
        ],
    )
    def sc_kernel(x_r, sd_r, z2_r, z1_r, ones_r,
                  out_sum, out_deg,
                  sd_v, rows, ones_v, deg_v, acc, dacc, isem, gsem, ssem, dsem):
        c = lax.axis_index("c")
        t = lax.axis_index("s")
        chunks = []
        off = 0
        while off < rows_pt:
            chunks.append((off, min(GROUP, rows_pt - off)))
            off += GROUP

        def run(x_half, out_half, do_deg):
            base_r = t * rows_pt
            pltpu.sync_copy(z2_r, rows.at[0])
            for (o, sz) in chunks:
                pltpu.sync_copy(rows.at[0].at[pl.ds(0, sz)],
                                acc.at[pl.ds(base_r + o, sz)])
            if do_deg:
                pltpu.sync_copy(ones_r, ones_v)
                pltpu.sync_copy(z1_r, deg_v)
                pltpu.sync_copy(deg_v, dacc.at[pl.ds(base_r, rows_pt)])
            for j in range(NIDX):
                pltpu.async_copy(sd_r.at[t, j], sd_v.at[j], isem.at[j])
            for b in range(NBUF):
                pltpu.make_async_copy(sd_r.at[t, b], sd_v.at[b],
                                      isem.at[b]).wait()
                pltpu.async_copy(x_half.at[sd_v.at[b, 0]], rows.at[b],
                                 gsem.at[b])
            plsc.subcore_barrier()

            @pl.loop(0, g_steps, step=NBUF)
            def _(g0):
                for b in range(NBUF):
                    g = g0 + b
                    j = lax.rem(g, NIDX)
                    pltpu.make_async_copy(x_half.at[sd_v.at[j, 0]],
                                          rows.at[b], gsem.at[b]).wait()
                    pltpu.async_copy(rows.at[b], acc.at[sd_v.at[j, 1]],
                                     ssem.at[b], add=True)
                    if do_deg:
                        pltpu.async_copy(ones_v, dacc.at[sd_v.at[j, 1]],
                                         dsem.at[b], add=True)
                for b in range(NBUF):
                    g = g0 + b
                    gn = g + NBUF
                    h = g + NIDX

                    @pl.when(gn < g_steps)
                    def _():
                        j = lax.rem(g, NIDX)
                        jn = lax.rem(gn, NIDX)
                        pltpu.make_async_copy(rows.at[b],
                                              acc.at[sd_v.at[j, 1]],
                                              ssem.at[b]).wait()
                        if do_deg:
                            pltpu.make_async_copy(ones_v,
                                                  dacc.at[sd_v.at[j, 1]],
                                                  dsem.at[b]).wait()

                        @pl.when(h < g_steps)
                        def _():
                            pltpu.async_copy(sd_r.at[t, h], sd_v.at[j],
                                             isem.at[j])

                        pltpu.make_async_copy(sd_r.at[t, gn], sd_v.at[jn],
                                              isem.at[jn]).wait()
                        pltpu.async_copy(x_half.at[sd_v.at[jn, 0]],
                                         rows.at[b], gsem.at[b])

            for b in range(NBUF):
                pltpu.make_async_copy(rows.at[b], acc.at[sd_v.at[0, 1]],
                                      ssem.at[b]).wait()
                if do_deg:
                    pltpu.make_async_copy(ones_v, dacc.at[sd_v.at[0, 1]],
                                          dsem.at[b]).wait()
            plsc.subcore_barrier()
            for i, (o, sz) in enumerate(chunks):
                pltpu.sync_copy(acc.at[pl.ds(base_r + o, sz)],
                                rows.at[i % NBUF].at[pl.ds(0, sz)])
                pltpu.sync_copy(rows.at[i % NBUF].at[pl.ds(0, sz)],
                                out_half.at[pl.ds(base_r + o, sz)])
            if do_deg:
                pltpu.sync_copy(dacc.at[pl.ds(base_r, rows_pt)], deg_v)
                pltpu.sync_copy(deg_v, out_deg.at[pl.ds(base_r, rows_pt)])

        @pl.when(c == 0)
        def _():
            run(x_r.at[:, pl.ds(0, half)], out_sum.at[0], True)

        @pl.when(c == 1)
        def _():
            run(x_r.at[:, pl.ds(half, half)], out_sum.at[1], False)

    return sc_kernel(x, sd, z2, z1, ones)


def _tc_combine(x, sum2, deg, wst, wnlo, wnhi, b2, blk):
    n, d_in = x.shape
    d_out = wst.shape[1]
    half = sum2.shape[2]

    def body(x_ref, lo_ref, hi_ref, deg_ref, wst_ref, wnlo_ref, wnhi_ref,
             b_ref, out_ref):
        r = 1.0 / jnp.maximum(deg_ref[...], 1.0)
        acc = jnp.dot(x_ref[...], wst_ref[...],
                      preferred_element_type=jnp.float32)
        nb = jnp.dot(lo_ref[0], wnlo_ref[...],
                     preferred_element_type=jnp.float32)
        nb = nb + jnp.dot(hi_ref[0], wnhi_ref[...],
                          preferred_element_type=jnp.float32)
        out_ref[...] = acc + nb * r + b_ref[...]

    return pl.pallas_call(
        body,
        grid=(n // blk,),
        in_specs=[
            pl.BlockSpec((blk, d_in), lambda i: (i, 0)),
            pl.BlockSpec((1, blk, half), lambda i: (0, i, 0)),
            pl.BlockSpec((1, blk, half), lambda i: (1, i, 0)),
            pl.BlockSpec((blk, 1), lambda i: (i, 0)),
            pl.BlockSpec((d_in, d_out), lambda i: (0, 0)),
            pl.BlockSpec((half, d_out), lambda i: (0, 0)),
            pl.BlockSpec((half, d_out), lambda i: (0, 0)),
            pl.BlockSpec((1, d_out), lambda i: (0, 0)),
        ],
        out_specs=pl.BlockSpec((blk, d_out), lambda i: (i, 0)),
        out_shape=jax.ShapeDtypeStruct((n, d_out), jnp.float32),
    )(x, sum2, sum2, deg, wst, wnlo, wnhi, b2)


def kernel(x, edge_index, W_self, W_neigh, b):
    n, d_in = x.shape
    e = edge_index.shape[1]
    half = d_in // 2

    ept = -(-e // (NTILES * GROUP * NBUF)) * (GROUP * NBUF)
    e_pad = NTILES * ept
    g_steps = ept // GROUP
    n_rows = (n // (NTILES * 16) + 1) * (NTILES * 16)
    n_dummy = n_rows - n

    src = edge_index[0].astype(jnp.int32)
    dst = edge_index[1].astype(jnp.int32)
    pad_ar = jnp.arange(e_pad - e, dtype=jnp.int32)
    srcr = jnp.concatenate([src, pad_ar % n]).reshape(NTILES, g_steps, GROUP)
    dstr = jnp.concatenate([dst, n + pad_ar % n_dummy]).reshape(
        NTILES, g_steps, GROUP)
    sd = jnp.stack([srcr, dstr], axis=2)

    z2 = jnp.zeros((GROUP, half), jnp.float32)
    z1 = jnp.zeros((n_rows // NTILES,), jnp.float32)
    ones = jnp.ones((GROUP,), jnp.float32)

    wst = W_self.T
    wnt = W_neigh.T
    wnlo = wnt[:half]
    wnhi = wnt[half:]
    b2 = b.reshape(1, -1)
    sum2, deg = _sc_aggregate(x, sd, z2, z1, ones, n_rows, half)
    deg = deg.reshape(n_rows, 1)
    return _tc_combine(x, sum2, deg, wst, wnlo, wnhi, b2, blk=1000)
